# baseline (speedup 1.0000x reference)
import jax
import jax.numpy as jnp
from jax import lax
from jax.experimental import pallas as pl
from jax.experimental.pallas import tpu as pltpu
from jax.experimental.pallas import tpu_sc as plsc

_NC = 2
_NS = 16
_NW = _NC * _NS
_CH = 128
_NBUF = 4


def kernel(tc, embedding):
    batch, hist = tc.shape
    n_rows, depth = embedding.shape
    width = 2 * depth
    n_idx = batch * hist
    n_pairs = n_idx // 2
    per_worker = n_pairs // _NW
    n_chunks = per_worker // _CH
    assert per_worker % _CH == 0 and (n_chunks - _NBUF) % _NBUF == 0

    def build_body(emb_ref, out_ref):
        i = pl.program_id(0)
        left = jnp.broadcast_to(emb_ref[i, :][None, :], (n_rows, depth))
        out_ref[0] = jnp.concatenate([left, emb_ref[...]], axis=-1)

    pair_table = pl.pallas_call(
        build_body,
        grid=(n_rows,),
        in_specs=[pl.BlockSpec((n_rows, depth), lambda i: (0, 0))],
        out_specs=pl.BlockSpec((1, n_rows, width), lambda i: (i, 0, 0)),
        out_shape=jax.ShapeDtypeStruct((n_rows, n_rows, width),
                                       embedding.dtype),
    )(embedding).reshape(n_rows * n_rows, width)

    flat = tc.reshape(n_pairs, 2)
    pair_idx = flat[:, 0] * n_rows + flat[:, 1]

    mesh = plsc.VectorSubcoreMesh(core_axis_name="core",
                                  subcore_axis_name="subcore")

    scratch = (
        [pltpu.VMEM((_CH,), jnp.int32) for _ in range(_NBUF)]
        + [pltpu.VMEM((_CH, width), jnp.float32) for _ in range(_NBUF)]
        + [pltpu.SemaphoreType.DMA for _ in range(2 * _NBUF)]
    )

    @pl.kernel(
        out_type=jax.ShapeDtypeStruct((n_pairs, width), embedding.dtype),
        mesh=mesh,
        scratch_types=scratch,
    )
    def gather_kernel(table_hbm, i_hbm, o_hbm, *bufs):
        idx_v = bufs[:_NBUF]
        rows_v = bufs[_NBUF:2 * _NBUF]
        gsem = bufs[2 * _NBUF:3 * _NBUF]
        ssem = bufs[3 * _NBUF:]

        wid = lax.axis_index("subcore") * _NC + lax.axis_index("core")
        base = wid * per_worker

        def load_and_gather(b, k):
            pltpu.sync_copy(i_hbm.at[pl.ds(base + k * _CH, _CH)], idx_v[b])
            pltpu.make_async_copy(
                table_hbm.at[idx_v[b]], rows_v[b], gsem[b]).start()

        def complete(b, k):
            pltpu.make_async_copy(
                table_hbm.at[idx_v[b]], rows_v[b], gsem[b]).wait()
            store = pltpu.make_async_copy(
                rows_v[b], o_hbm.at[pl.ds(base + k * _CH, _CH)], ssem[b])
            store.start()
            store.wait()

        for b in range(_NBUF):
            load_and_gather(b, b)

        @pl.loop(_NBUF, n_chunks, step=_NBUF)
        def _(k0):
            for b in range(_NBUF):
                complete(b, k0 + b - _NBUF)
                load_and_gather(b, k0 + b)

        for b in range(_NBUF):
            complete(b, n_chunks - _NBUF + b)

    out = gather_kernel(pair_table, pair_idx)
    return out.reshape(batch, hist, depth)


# speedup vs baseline: 1.8183x; 1.8183x over previous
import jax
import jax.numpy as jnp
from jax import lax
from jax.experimental import pallas as pl
from jax.experimental.pallas import tpu as pltpu
from jax.experimental.pallas import tpu_sc as plsc

_NC = 2
_NS = 16
_NW = _NC * _NS
_CH = 80
_NBUF = 4
_VL = 16


def kernel(tc, embedding):
    batch, hist = tc.shape
    n_rows, depth = embedding.shape
    width = 2 * depth
    n_idx = batch * hist
    n_pairs = n_idx // 2
    half_batch = batch // 2
    per_worker = n_pairs // _NW
    n_chunks = per_worker // _CH
    assert per_worker % _CH == 0 and (n_chunks - _NBUF) % _NBUF == 0

    def build_body(emb_ref, out_ref):
        i = pl.program_id(0)
        left = jnp.broadcast_to(emb_ref[i, :][None, :], (n_rows, depth))
        out_ref[0] = jnp.concatenate([left, emb_ref[...]], axis=-1)

    pair_table = pl.pallas_call(
        build_body,
        grid=(n_rows,),
        in_specs=[pl.BlockSpec((n_rows, depth), lambda i: (0, 0))],
        out_specs=pl.BlockSpec((1, n_rows, width), lambda i: (i, 0, 0)),
        out_shape=jax.ShapeDtypeStruct((n_rows, n_rows, width),
                                       embedding.dtype),
    )(embedding).reshape(n_rows * n_rows, width)

    pair_idx = (tc[:half_batch] * n_rows + tc[half_batch:]).reshape(n_pairs)

    mesh = plsc.VectorSubcoreMesh(core_axis_name="core",
                                  subcore_axis_name="subcore")

    scratch = (
        [pltpu.VMEM((_CH,), jnp.int32) for _ in range(_NBUF)]
        + [pltpu.VMEM((_CH, width), jnp.float32) for _ in range(_NBUF)]
        + [pltpu.VMEM((_CH, depth), jnp.float32) for _ in range(2 * _NBUF)]
        + [pltpu.SemaphoreType.DMA for _ in range(3 * _NBUF)]
    )

    @pl.kernel(
        out_type=jax.ShapeDtypeStruct((n_idx, depth), embedding.dtype),
        mesh=mesh,
        scratch_types=scratch,
    )
    def gather_kernel(table_hbm, i_hbm, o_hbm, *bufs):
        idx_v = bufs[:_NBUF]
        rows_v = bufs[_NBUF:2 * _NBUF]
        left_v = bufs[2 * _NBUF:3 * _NBUF]
        right_v = bufs[3 * _NBUF:4 * _NBUF]
        gsem = bufs[4 * _NBUF:5 * _NBUF]
        ssem_l = bufs[5 * _NBUF:6 * _NBUF]
        ssem_r = bufs[6 * _NBUF:]

        wid = lax.axis_index("subcore") * _NC + lax.axis_index("core")
        base = wid * per_worker

        def load_and_gather(b, k):
            pltpu.sync_copy(i_hbm.at[pl.ds(base + k * _CH, _CH)], idx_v[b])
            pltpu.make_async_copy(
                table_hbm.at[idx_v[b]], rows_v[b], gsem[b]).start()

        def complete(b, k):
            pltpu.make_async_copy(
                table_hbm.at[idx_v[b]], rows_v[b], gsem[b]).wait()

            @pl.loop(0, _CH)
            def _(r):
                for c in range(depth // _VL):
                    sl = pl.ds(c * _VL, _VL)
                    sr = pl.ds(depth + c * _VL, _VL)
                    left_v[b][r, sl] = rows_v[b][r, sl]
                    right_v[b][r, sl] = rows_v[b][r, sr]

            s = base + k * _CH
            left = pltpu.make_async_copy(
                left_v[b], o_hbm.at[pl.ds(s, _CH)], ssem_l[b])
            right = pltpu.make_async_copy(
                right_v[b], o_hbm.at[pl.ds(s + n_pairs, _CH)], ssem_r[b])
            left.start()
            right.start()
            left.wait()
            right.wait()

        for b in range(_NBUF):
            load_and_gather(b, b)

        @pl.loop(_NBUF, n_chunks, step=_NBUF)
        def _(k0):
            for b in range(_NBUF):
                complete(b, k0 + b - _NBUF)
                load_and_gather(b, k0 + b)

        for b in range(_NBUF):
            complete(b, n_chunks - _NBUF + b)

    out = gather_kernel(pair_table, pair_idx)
    return out.reshape(batch, hist, depth)


# speedup vs baseline: 2.0002x; 1.1000x over previous
import jax
import jax.numpy as jnp
from jax import lax
from jax.experimental import pallas as pl
from jax.experimental.pallas import tpu as pltpu
from jax.experimental.pallas import tpu_sc as plsc

_NC = 2
_NS = 16
_NW = _NC * _NS
_CH = 80
_NBUF = 4
_VL = 16


def kernel(tc, embedding):
    batch, hist = tc.shape
    n_rows, depth = embedding.shape
    width = 2 * depth
    n_idx = batch * hist
    n_pairs = n_idx // 2
    half_batch = batch // 2
    per_worker = n_pairs // _NW
    n_chunks = per_worker // _CH
    assert per_worker % _CH == 0 and (n_chunks - _NBUF) % _NBUF == 0

    def build_body(emb_ref, out_ref):
        i = pl.program_id(0)
        left = jnp.broadcast_to(emb_ref[i, :][None, :], (n_rows, depth))
        out_ref[0] = jnp.concatenate([left, emb_ref[...]], axis=-1)

    pair_table = pl.pallas_call(
        build_body,
        grid=(n_rows,),
        in_specs=[pl.BlockSpec((n_rows, depth), lambda i: (0, 0))],
        out_specs=pl.BlockSpec((1, n_rows, width), lambda i: (i, 0, 0)),
        out_shape=jax.ShapeDtypeStruct((n_rows, n_rows, width),
                                       embedding.dtype),
    )(embedding).reshape(n_rows * n_rows, width)

    pair_idx = (tc[:half_batch] * n_rows + tc[half_batch:]).reshape(n_pairs)

    mesh = plsc.VectorSubcoreMesh(core_axis_name="core",
                                  subcore_axis_name="subcore")

    scratch = (
        [pltpu.VMEM((_CH,), jnp.int32) for _ in range(_NBUF)]
        + [pltpu.VMEM((_CH, width), jnp.float32) for _ in range(_NBUF)]
        + [pltpu.VMEM((_CH, depth), jnp.float32) for _ in range(2 * _NBUF)]
        + [pltpu.SemaphoreType.DMA for _ in range(3 * _NBUF)]
    )

    @pl.kernel(
        out_type=jax.ShapeDtypeStruct((n_idx, depth), embedding.dtype),
        mesh=mesh,
        scratch_types=scratch,
    )
    def gather_kernel(table_hbm, i_hbm, o_hbm, *bufs):
        idx_v = bufs[:_NBUF]
        rows_v = bufs[_NBUF:2 * _NBUF]
        left_v = bufs[2 * _NBUF:3 * _NBUF]
        right_v = bufs[3 * _NBUF:4 * _NBUF]
        gsem = bufs[4 * _NBUF:5 * _NBUF]
        ssem_l = bufs[5 * _NBUF:6 * _NBUF]
        ssem_r = bufs[6 * _NBUF:]

        wid = lax.axis_index("subcore") * _NC + lax.axis_index("core")
        base = wid * per_worker

        def load_and_gather(b, k):
            pltpu.sync_copy(i_hbm.at[pl.ds(base + k * _CH, _CH)], idx_v[b])
            pltpu.make_async_copy(
                table_hbm.at[idx_v[b]], rows_v[b], gsem[b]).start()

        def store_handles(b, k):
            s = base + k * _CH
            return (
                pltpu.make_async_copy(
                    left_v[b], o_hbm.at[pl.ds(s, _CH)], ssem_l[b]),
                pltpu.make_async_copy(
                    right_v[b], o_hbm.at[pl.ds(s + n_pairs, _CH)],
                    ssem_r[b]),
            )

        def process(b, kc, wait_stores, start_next):
            pltpu.make_async_copy(
                table_hbm.at[idx_v[b]], rows_v[b], gsem[b]).wait()
            if wait_stores:
                prev_l, prev_r = store_handles(b, kc - _NBUF)
                prev_l.wait()
                prev_r.wait()

            @pl.loop(0, _CH, step=2)
            def _(r):
                for dr in range(2):
                    for c in range(depth // _VL):
                        sl = pl.ds(c * _VL, _VL)
                        sr = pl.ds(depth + c * _VL, _VL)
                        left_v[b][r + dr, sl] = rows_v[b][r + dr, sl]
                        right_v[b][r + dr, sl] = rows_v[b][r + dr, sr]

            cur_l, cur_r = store_handles(b, kc)
            cur_l.start()
            cur_r.start()
            if start_next:
                load_and_gather(b, kc + _NBUF)

        for b in range(_NBUF):
            load_and_gather(b, b)

        for b in range(_NBUF):
            process(b, b, wait_stores=False, start_next=True)

        @pl.loop(_NBUF, n_chunks - _NBUF, step=_NBUF)
        def _(k0):
            for b in range(_NBUF):
                process(b, k0 + b, wait_stores=True, start_next=True)

        for b in range(_NBUF):
            process(b, n_chunks - _NBUF + b, wait_stores=True,
                    start_next=False)
        for b in range(_NBUF):
            final_l, final_r = store_handles(b, n_chunks - _NBUF + b)
            final_l.wait()
            final_r.wait()

    out = gather_kernel(pair_table, pair_idx)
    return out.reshape(batch, hist, depth)


# speedup vs baseline: 2.0080x; 1.0039x over previous
import jax
import jax.numpy as jnp
from jax import lax
from jax.experimental import pallas as pl
from jax.experimental.pallas import tpu as pltpu
from jax.experimental.pallas import tpu_sc as plsc

_NC = 2
_NS = 16
_NW = _NC * _NS
_CH = 80
_NBUF = 4
_VL = 16


def kernel(tc, embedding):
    batch, hist = tc.shape
    n_rows, depth = embedding.shape
    width = 2 * depth
    n_idx = batch * hist
    n_pairs = n_idx // 2
    half_batch = batch // 2
    per_worker = n_pairs // _NW
    n_chunks = per_worker // _CH
    assert per_worker % _CH == 0 and (n_chunks - _NBUF) % _NBUF == 0

    def build_body(emb_ref, out_ref):
        i = pl.program_id(0)
        left = jnp.broadcast_to(emb_ref[i, :][None, :], (n_rows, depth))
        out_ref[0] = jnp.concatenate([left, emb_ref[...]], axis=-1)

    pair_table = pl.pallas_call(
        build_body,
        grid=(n_rows,),
        in_specs=[pl.BlockSpec((n_rows, depth), lambda i: (0, 0))],
        out_specs=pl.BlockSpec((1, n_rows, width), lambda i: (i, 0, 0)),
        out_shape=jax.ShapeDtypeStruct((n_rows, n_rows, width),
                                       embedding.dtype),
    )(embedding).reshape(n_rows * n_rows, width)

    pair_idx = (tc[:half_batch] * n_rows + tc[half_batch:]).reshape(n_pairs)

    mesh = plsc.VectorSubcoreMesh(core_axis_name="core",
                                  subcore_axis_name="subcore")

    _NI = 2 * _NBUF
    scratch = (
        [pltpu.VMEM((_CH,), jnp.int32) for _ in range(_NI)]
        + [pltpu.VMEM((_CH, width), jnp.float32) for _ in range(_NBUF)]
        + [pltpu.VMEM((_CH, depth), jnp.float32) for _ in range(2 * _NBUF)]
        + [pltpu.SemaphoreType.DMA for _ in range(3 * _NBUF + _NI)]
    )

    @pl.kernel(
        out_type=jax.ShapeDtypeStruct((n_idx, depth), embedding.dtype),
        mesh=mesh,
        scratch_types=scratch,
    )
    def gather_kernel(table_hbm, i_hbm, o_hbm, *bufs):
        idx_v = bufs[:_NI]
        rows_v = bufs[_NI:_NI + _NBUF]
        left_v = bufs[_NI + _NBUF:_NI + 2 * _NBUF]
        right_v = bufs[_NI + 2 * _NBUF:_NI + 3 * _NBUF]
        gsem = bufs[_NI + 3 * _NBUF:_NI + 4 * _NBUF]
        ssem_l = bufs[_NI + 4 * _NBUF:_NI + 5 * _NBUF]
        ssem_r = bufs[_NI + 5 * _NBUF:_NI + 6 * _NBUF]
        isem = bufs[_NI + 6 * _NBUF:]

        wid = lax.axis_index("subcore") * _NC + lax.axis_index("core")
        base = wid * per_worker

        def idx_handle(slot, k):
            return pltpu.make_async_copy(
                i_hbm.at[pl.ds(base + k * _CH, _CH)], idx_v[slot],
                isem[slot])

        def store_handles(b, k):
            s = base + k * _CH
            return (
                pltpu.make_async_copy(
                    left_v[b], o_hbm.at[pl.ds(s, _CH)], ssem_l[b]),
                pltpu.make_async_copy(
                    right_v[b], o_hbm.at[pl.ds(s + n_pairs, _CH)],
                    ssem_r[b]),
            )

        def process(b, islot, kc, wait_stores, prefetch, start_next):
            pltpu.make_async_copy(
                table_hbm.at[idx_v[islot]], rows_v[b], gsem[b]).wait()
            if prefetch:
                idx_handle(islot, kc + _NI).start()
            if wait_stores:
                prev_l, prev_r = store_handles(b, kc - _NBUF)
                prev_l.wait()
                prev_r.wait()

            @pl.loop(0, _CH, step=2)
            def _(r):
                for dr in range(2):
                    for c in range(depth // _VL):
                        sl = pl.ds(c * _VL, _VL)
                        sr = pl.ds(depth + c * _VL, _VL)
                        left_v[b][r + dr, sl] = rows_v[b][r + dr, sl]
                        right_v[b][r + dr, sl] = rows_v[b][r + dr, sr]

            cur_l, cur_r = store_handles(b, kc)
            cur_l.start()
            cur_r.start()
            if start_next:
                nslot = (islot + _NBUF) % _NI
                idx_handle(nslot, kc + _NBUF).wait()
                pltpu.make_async_copy(
                    table_hbm.at[idx_v[nslot]], rows_v[b], gsem[b]).start()

        def visit(k0, phase, wait_stores, prefetch, start_next):
            for b in range(_NBUF):
                process(b, phase * _NBUF + b, k0 + b, wait_stores,
                        prefetch, start_next)

        for k in range(_NI):
            idx_handle(k, k).start()
        for b in range(_NBUF):
            idx_handle(b, b).wait()
            pltpu.make_async_copy(
                table_hbm.at[idx_v[b]], rows_v[b], gsem[b]).start()

        visit(0, 0, wait_stores=False, prefetch=True, start_next=True)
        visit(_NBUF, 1, wait_stores=True, prefetch=True, start_next=True)

        @pl.loop(_NI, n_chunks - 2 * _NI, step=_NI)
        def _(k0):
            visit(k0, 0, wait_stores=True, prefetch=True, start_next=True)
            visit(k0 + _NBUF, 1, wait_stores=True, prefetch=True,
                  start_next=True)

        tail = n_chunks - 2 * _NI
        visit(tail, 0, wait_stores=True, prefetch=True, start_next=True)
        visit(tail + _NBUF, 1, wait_stores=True, prefetch=True,
              start_next=True)
        visit(tail + _NI, 0, wait_stores=True, prefetch=False,
              start_next=True)
        visit(tail + _NI + _NBUF, 1, wait_stores=True, prefetch=False,
              start_next=False)
        for b in range(_NBUF):
            final_l, final_r = store_handles(b, n_chunks - _NBUF + b)
            final_l.wait()
            final_r.wait()

    out = gather_kernel(pair_table, pair_idx)
    return out.reshape(batch, hist, depth)
